# Initial kernel scaffold; baseline (speedup 1.0000x reference)
import jax
import jax.numpy as jnp
from jax.experimental import pallas as pl


def kernel(feat_a, feat_b, dirs_a, dirs_b, Wq, bq, Wk, bk):
    raise NotImplementedError("write your pallas kernel here")


# speedup vs baseline: 28.9660x; 28.9660x over previous
import math

import jax
import jax.numpy as jnp
import numpy as np
from jax.experimental import pallas as pl
from jax.experimental.pallas import tpu as pltpu

B = 2
N = 2048
DIM = 768
TOPK_COARSE = 64
TOPK_FINE = 16
TEMPERATURE = 0.07
ROWS = 256

_COS_THRESH = np.float32(np.cos(np.float64(np.float32(math.radians(15.0)))))
_INT_MIN = np.int32(-2147483648)


def _kth_largest(x, k):
    rows = x.shape[0]
    cnt0 = jnp.sum((x >= 0).astype(jnp.int32), axis=1, keepdims=True)
    prefix = jnp.where(cnt0 >= k, jnp.int32(0), jnp.int32(_INT_MIN))
    prefix = jnp.broadcast_to(prefix, (rows, 1))
    for bit in range(30, -1, -1):
        cand = prefix + jnp.int32(1 << bit)
        cnt = jnp.sum((x >= cand).astype(jnp.int32), axis=1, keepdims=True)
        prefix = jnp.where(cnt >= k, cand, prefix)
    return prefix


def _body(fa, fb, da, dbT, wq, bq, wk, bk, out, k_scr):
    i = pl.program_id(1)

    @pl.when(i == 0)
    def _():
        k_scr[...] = jax.lax.dot_general(
            fb[0].astype(jnp.bfloat16), wk[...].astype(jnp.bfloat16),
            (((1,), (1,)), ((), ())),
            preferred_element_type=jnp.float32) + bk[...]

    q = jax.lax.dot_general(
        fa[0].astype(jnp.bfloat16), wq[...].astype(jnp.bfloat16),
        (((1,), (1,)), ((), ())),
        preferred_element_type=jnp.float32) + bq[...]
    k = k_scr[...]
    sim = jax.lax.dot_general(q, k, (((1,), (1,)), ((), ())),
                              preferred_element_type=jnp.float32,
                              precision=jax.lax.Precision.HIGHEST)
    sim_sel = jax.lax.dot_general(q.astype(jnp.bfloat16), k.astype(jnp.bfloat16),
                                  (((1,), (1,)), ((), ())),
                                  preferred_element_type=jnp.float32)

    dav = da[0]
    dbv = dbT[0]
    cos = (dav[:, 0:1] * dbv[0:1, :] + dav[:, 1:2] * dbv[1:2, :]
           + dav[:, 2:3] * dbv[2:3, :])
    mask = cos >= _COS_THRESH

    bits = jax.lax.bitcast_convert_type(sim, jnp.int32)
    sortable = jnp.where(bits >= 0, bits, bits ^ jnp.int32(0x7FFFFFFF))
    bits_sel = jax.lax.bitcast_convert_type(sim_sel, jnp.int32)
    sortable_sel = jnp.where(bits_sel >= 0, bits_sel,
                             bits_sel ^ jnp.int32(0x7FFFFFFF))

    t64 = _kth_largest(sortable_sel, TOPK_COARSE)
    in_a = sortable_sel >= t64
    keep_m = in_a & mask
    cnt_m = jnp.sum(keep_m.astype(jnp.int32), axis=1, keepdims=True)
    keep = in_a & (mask | (cnt_m == 0))
    z = jnp.where(keep, sortable, _INT_MIN)
    t16 = _kth_largest(z, TOPK_FINE)
    sel = keep & (z >= t16)

    logits = sim / TEMPERATURE
    lsel = jnp.where(sel, logits, -jnp.inf)
    lmax = jnp.max(lsel, axis=1, keepdims=True)
    e = jnp.exp(lsel - lmax)
    s = jnp.sum(e, axis=1, keepdims=True)
    probs = e / s
    s2 = jnp.sum(probs, axis=1, keepdims=True)
    out[0] = probs / (s2 + jnp.float32(1e-8))


def kernel(feat_a, feat_b, dirs_a, dirs_b, Wq, bq, Wk, bk):
    dbT = jnp.swapaxes(dirs_b, 1, 2)
    bq2 = bq.reshape(1, DIM)
    bk2 = bk.reshape(1, DIM)
    grid = (B, N // ROWS)
    return pl.pallas_call(
        _body,
        grid=grid,
        in_specs=[
            pl.BlockSpec((1, ROWS, DIM), lambda b, i: (b, i, 0)),
            pl.BlockSpec((1, N, DIM), lambda b, i: (b, 0, 0)),
            pl.BlockSpec((1, ROWS, 3), lambda b, i: (b, i, 0)),
            pl.BlockSpec((1, 3, N), lambda b, i: (b, 0, 0)),
            pl.BlockSpec((DIM, DIM), lambda b, i: (0, 0)),
            pl.BlockSpec((1, DIM), lambda b, i: (0, 0)),
            pl.BlockSpec((DIM, DIM), lambda b, i: (0, 0)),
            pl.BlockSpec((1, DIM), lambda b, i: (0, 0)),
        ],
        out_specs=pl.BlockSpec((1, ROWS, N), lambda b, i: (b, i, 0)),
        out_shape=jax.ShapeDtypeStruct((B, N, N), jnp.float32),
        scratch_shapes=[pltpu.VMEM((N, DIM), jnp.float32)],
    )(feat_a, feat_b, dirs_a, dbT, Wq, bq2, Wk, bk2)
